# baseline (speedup 1.0000x reference)
import numpy as np
import jax
import jax.numpy as jnp
from jax.experimental import pallas as pl

_RATIO = 0.7
_DIM_OUT = 128
_N_CENTER = 64
_KS = 16
_RADIUS = 0.4
_SIGMA = 0.1
_KA = 12
_M = 2048
_B = 2


def _fib_sphere(n, r):
    i = np.arange(n, dtype=np.float64)
    phi = np.pi * (3.0 - np.sqrt(5.0))
    y = 1.0 - 2.0 * (i + 0.5) / n
    rad = np.sqrt(np.maximum(0.0, 1.0 - y * y))
    th = phi * i
    return (np.stack([np.cos(th) * rad, y, np.sin(th) * rad], axis=-1) * r).astype(np.float32)


def _mk_anchors(n):
    rng = np.random.RandomState(0)
    out = []
    for _ in range(n):
        a = rng.randn(3, 3)
        q, rmat = np.linalg.qr(a)
        q = q * np.sign(np.diag(rmat))[None, :]
        if np.linalg.det(q) < 0:
            q[:, 0] = -q[:, 0]
        out.append(q)
    return np.stack(out).astype(np.float32)


_KPTS = _fib_sphere(_KS, _RATIO * _RADIUS)
_ANCHORS_NP = _mk_anchors(_KA)
_KERNELS_NP = np.transpose(_ANCHORS_NP @ _KPTS.T, (2, 0, 1))
_KCOL_NP = np.transpose(_KERNELS_NP, (1, 0, 2)).reshape(_KA * _KS, 3)
_K2_NP = np.sum(_KCOL_NP * _KCOL_NP, axis=-1)


def _body(frag_ref, clouds_ref, w_ref, kcolT_ref, k2_ref, out_ref):
    inv_s = 1.0 / _SIGMA
    inv_2s = 1.0 / (2.0 * _SIGMA)
    frag = frag_ref[:]
    fragT = jnp.transpose(frag)
    kcolT = kcolT_ref[:]
    C = jnp.concatenate([jnp.transpose(clouds_ref[0]),
                         jnp.transpose(clouds_ref[1])], axis=0)

    FK = (frag[:, 0:1] * kcolT[0:1, :]
          + frag[:, 1:2] * kcolT[1:2, :]
          + frag[:, 2:3] * kcolT[2:3, :])
    E = jnp.exp(FK * inv_s)

    d0 = fragT[0:1, :] - C[:, 0:1]
    d1 = fragT[1:2, :] - C[:, 1:2]
    d2_ = fragT[2:3, :] - C[:, 2:3]
    d2c = d0 * d0 + d1 * d1 + d2_ * d2_
    mask = d2c < (_RADIUS * _RADIUS)
    Mw = jnp.where(mask, jnp.exp(d2c * (-inv_2s)), 0.0)
    nn = jnp.sum(jnp.where(mask, 1.0, 0.0), axis=1, keepdims=True)

    S = jax.lax.dot_general(Mw, E, (((1,), (0,)), ((), ())),
                            preferred_element_type=jnp.float32,
                            precision=jax.lax.Precision.HIGHEST)

    CK = (C[:, 0:1] * kcolT[0:1, :]
          + C[:, 1:2] * kcolT[1:2, :]
          + C[:, 2:3] * kcolT[2:3, :])
    g = jnp.exp(k2_ref[:] * (-inv_2s) - CK * inv_s)
    Ss = S * g / (nn + 1.0)

    SsT = jnp.transpose(Ss)
    for b in range(_B):
        SsTb = SsT[:, b * _N_CENTER:(b + 1) * _N_CENTER]
        Yb = jnp.stack(
            [SsTb[a * _KS:(a + 1) * _KS, :] for a in range(_KA)],
            axis=-1).reshape(_KS, _N_CENTER * _KA)
        out_ref[b] = jax.lax.dot_general(
            w_ref[:], Yb, (((1,), (0,)), ((), ())),
            preferred_element_type=jnp.float32,
            precision=jax.lax.Precision.HIGHEST)


def kernel(frag, clouds, W):
    kcolT = jnp.asarray(_KCOL_NP.T)
    k2 = jnp.asarray(_K2_NP)[None, :]

    F = pl.pallas_call(
        _body,
        out_shape=jax.ShapeDtypeStruct((_B, _DIM_OUT, _N_CENTER * _KA), jnp.float32),
    )(frag, clouds, W, kcolT, k2)

    feats = F.reshape(_B, _DIM_OUT, _N_CENTER, _KA)
    return clouds, feats, jnp.asarray(_ANCHORS_NP)


# speedup vs baseline: 1.3030x; 1.3030x over previous
import numpy as np
import jax
import jax.numpy as jnp
from jax.experimental import pallas as pl

_RATIO = 0.7
_DIM_OUT = 128
_N_CENTER = 64
_KS = 16
_RADIUS = 0.4
_SIGMA = 0.1
_KA = 12
_M = 2048
_B = 2


def _fib_sphere(n, r):
    i = np.arange(n, dtype=np.float64)
    phi = np.pi * (3.0 - np.sqrt(5.0))
    y = 1.0 - 2.0 * (i + 0.5) / n
    rad = np.sqrt(np.maximum(0.0, 1.0 - y * y))
    th = phi * i
    return (np.stack([np.cos(th) * rad, y, np.sin(th) * rad], axis=-1) * r).astype(np.float32)


def _mk_anchors(n):
    rng = np.random.RandomState(0)
    out = []
    for _ in range(n):
        a = rng.randn(3, 3)
        q, rmat = np.linalg.qr(a)
        q = q * np.sign(np.diag(rmat))[None, :]
        if np.linalg.det(q) < 0:
            q[:, 0] = -q[:, 0]
        out.append(q)
    return np.stack(out).astype(np.float32)


_KPTS = _fib_sphere(_KS, _RATIO * _RADIUS)
_ANCHORS_NP = _mk_anchors(_KA)
_KERNELS_NP = np.transpose(_ANCHORS_NP @ _KPTS.T, (2, 0, 1))
_KCOL_NP = np.transpose(_KERNELS_NP, (1, 0, 2)).reshape(_KA * _KS, 3)
_K2_NP = np.sum(_KCOL_NP * _KCOL_NP, axis=-1)


def _body(frag_ref, clouds_ref, w_ref, kcolT_ref, k2_ref, out_ref):
    inv_s = 1.0 / _SIGMA
    inv_2s = 1.0 / (2.0 * _SIGMA)
    frag = frag_ref[:]
    kcolT = kcolT_ref[:]
    C = jnp.concatenate([jnp.transpose(clouds_ref[0]),
                         jnp.transpose(clouds_ref[1])], axis=0)
    CT = jnp.transpose(C)

    FK = (frag[:, 0:1] * kcolT[0:1, :]
          + frag[:, 1:2] * kcolT[1:2, :]
          + frag[:, 2:3] * kcolT[2:3, :])
    E = jnp.exp(FK * inv_s)

    d0 = frag[:, 0:1] - CT[0:1, :]
    d1 = frag[:, 1:2] - CT[1:2, :]
    d2_ = frag[:, 2:3] - CT[2:3, :]
    d2c = d0 * d0 + d1 * d1 + d2_ * d2_
    mask = d2c < (_RADIUS * _RADIUS)
    Mw = jnp.where(mask, jnp.exp(d2c * (-inv_2s)), 0.0)
    nnT = jnp.sum(jnp.where(mask, 1.0, 0.0), axis=0, keepdims=True)
    nn = jnp.transpose(nnT)

    S = jax.lax.dot_general(Mw, E, (((0,), (0,)), ((), ())),
                            preferred_element_type=jnp.float32)

    CK = (C[:, 0:1] * kcolT[0:1, :]
          + C[:, 1:2] * kcolT[1:2, :]
          + C[:, 2:3] * kcolT[2:3, :])
    g = jnp.exp(k2_ref[:] * (-inv_2s) - CK * inv_s)
    Ss = S * g / (nn + 1.0)

    wT = jnp.transpose(w_ref[:])
    for a in range(_KA):
        out_ref[:, a * _DIM_OUT:(a + 1) * _DIM_OUT] = jax.lax.dot_general(
            Ss[:, a * _KS:(a + 1) * _KS], wT, (((1,), (0,)), ((), ())),
            preferred_element_type=jnp.float32,
            precision=jax.lax.Precision.HIGHEST)


def kernel(frag, clouds, W):
    kcolT = jnp.asarray(_KCOL_NP.T)
    k2 = jnp.asarray(_K2_NP)[None, :]

    F = pl.pallas_call(
        _body,
        out_shape=jax.ShapeDtypeStruct((_B * _N_CENTER, _KA * _DIM_OUT), jnp.float32),
    )(frag, clouds, W, kcolT, k2)

    feats = F.reshape(_B, _N_CENTER, _KA, _DIM_OUT).transpose(0, 3, 1, 2)
    return clouds, feats, jnp.asarray(_ANCHORS_NP)
